# Initial kernel scaffold; baseline (speedup 1.0000x reference)
import jax
import jax.numpy as jnp
from jax.experimental import pallas as pl


def kernel(x, edge_index, edge_attr, batch, node_table, bond_tables, eps, mlp_W1, mlp_b1, mlp_bn_g, mlp_bn_b, mlp_bn_m, mlp_bn_v, mlp_W2, mlp_b2, bn_g, bn_b, bn_m, bn_v, vn_init, vn_W1, vn_b1, vn_bn1_g, vn_bn1_b, vn_bn1_m, vn_bn1_v, vn_W2, vn_b2, vn_bn2_g, vn_bn2_b, vn_bn2_m, vn_bn2_v, pred_W, pred_b):
    raise NotImplementedError("write your pallas kernel here")


# speedup vs baseline: 4.4233x; 4.4233x over previous
import functools

import jax
import jax.numpy as jnp
from jax import lax
from jax.experimental import pallas as pl
from jax.experimental.pallas import tpu as pltpu
from jax.experimental.pallas import tpu_sc as plsc

N = 10000
E = 320000
D = 128
H = 256
L = 5
G = 128
NC = 10

N_PAD = 10240
RB = 2048
NBLK = N_PAD // RB
NUM_SC = 2
NUM_TILES = 16
W_TILES = NUM_SC * NUM_TILES
E_TILE_CH = 79
E_TILE = E_TILE_CH * 128
E_PAD = W_TILES * E_TILE
ROWS_PER_TILE = N_PAD // NUM_TILES


def _onehot(ids, width):
  io = lax.broadcasted_iota(jnp.int32, (ids.shape[0], width), 1)
  return (ids[:, None] == io).astype(jnp.float32)


def _tc0_body(x_ref, b_ref, tab_ref, vn_ref, bond_ref, hv_ref, m_ref):
  xo = _onehot(x_ref[0, 0, :], 128)
  h = jnp.dot(xo, tab_ref[...], preferred_element_type=jnp.float32)
  bo = _onehot(b_ref[0, 0, :], G)
  hv = h + jnp.dot(bo, vn_ref[...], preferred_element_type=jnp.float32)
  hv_ref[...] = hv
  for a in range(4):
    m_ref[a] = jnp.maximum(hv + bond_ref[a][None, :], 0.0)


def _tc0(x3, batch3, tab, vn0, bond, interpret=False):
  return pl.pallas_call(
      _tc0_body,
      grid=(NBLK,),
      in_specs=[
          pl.BlockSpec((1, 1, RB), lambda i: (i, 0, 0)),
          pl.BlockSpec((1, 1, RB), lambda i: (i, 0, 0)),
          pl.BlockSpec((128, D), lambda i: (0, 0)),
          pl.BlockSpec((G, D), lambda i: (0, 0)),
          pl.BlockSpec((8, D), lambda i: (0, 0)),
      ],
      out_specs=[
          pl.BlockSpec((RB, D), lambda i: (i, 0)),
          pl.BlockSpec((4, RB, D), lambda i: (0, i, 0)),
      ],
      out_shape=[
          jax.ShapeDtypeStruct((N_PAD, D), jnp.float32),
          jax.ShapeDtypeStruct((4, N_PAD, D), jnp.float32),
      ],
      interpret=interpret,
  )(x3, batch3, tab, vn0, bond)


def _tc1_body(hv_ref, p_ref, b_ref, vn_ref, eps_ref, w1_ref, c1_ref, w2_ref,
              c2_ref, vw1_ref, vc1_ref, vw2_ref, vc2_ref,
              hnew_ref, vnnew_ref, s_ref):
  i = pl.program_id(0)

  @pl.when(i == 0)
  def _():
    s_ref[...] = jnp.zeros_like(s_ref)

  hv = hv_ref[...]
  z = eps_ref[0, 0] * hv + p_ref[0] + p_ref[1]
  t = jnp.maximum(
      jnp.dot(z, w1_ref[...], preferred_element_type=jnp.float32)
      + c1_ref[...], 0.0)
  zn = jnp.dot(t, w2_ref[...], preferred_element_type=jnp.float32) + c2_ref[...]
  hnew_ref[...] = jnp.maximum(zn, 0.0)
  bo = _onehot(b_ref[0, 0, :], G)
  s_ref[...] += lax.dot_general(bo, hv, (((0,), (0,)), ((), ())),
                                preferred_element_type=jnp.float32)

  @pl.when(i == NBLK - 1)
  def _():
    vt = s_ref[...] + vn_ref[...]
    vt = jnp.maximum(
        jnp.dot(vt, vw1_ref[...], preferred_element_type=jnp.float32)
        + vc1_ref[...], 0.0)
    vt = jnp.maximum(
        jnp.dot(vt, vw2_ref[...], preferred_element_type=jnp.float32)
        + vc2_ref[...], 0.0)
    vnnew_ref[...] = vt


def _tc1(hv, partials, batch3, vn, epsb, w1, c1, w2, c2, vw1, vc1, vw2, vc2,
         interpret=False):
  full = lambda shape: pl.BlockSpec(shape, lambda i: tuple(0 for _ in shape))
  return pl.pallas_call(
      _tc1_body,
      grid=(NBLK,),
      in_specs=[
          pl.BlockSpec((RB, D), lambda i: (i, 0)),
          pl.BlockSpec((2, RB, D), lambda i: (0, i, 0)),
          pl.BlockSpec((1, 1, RB), lambda i: (i, 0, 0)),
          full((G, D)), full((1, 128)),
          full((D, H)), full((1, H)), full((H, D)), full((1, D)),
          full((D, H)), full((1, H)), full((H, D)), full((1, D)),
      ],
      out_specs=[
          pl.BlockSpec((RB, D), lambda i: (i, 0)),
          pl.BlockSpec((G, D), lambda i: (0, 0)),
      ],
      out_shape=[
          jax.ShapeDtypeStruct((N_PAD, D), jnp.float32),
          jax.ShapeDtypeStruct((G, D), jnp.float32),
      ],
      scratch_shapes=[pltpu.VMEM((G, D), jnp.float32)],
      interpret=interpret,
  )(hv, partials, batch3, vn, epsb, w1, c1, w2, c2, vw1, vc1, vw2, vc2)


def _tc2_body(h_ref, b_ref, vn_ref, bond_ref, hv_ref, m_ref):
  bo = _onehot(b_ref[0, 0, :], G)
  hv = h_ref[...] + jnp.dot(bo, vn_ref[...], preferred_element_type=jnp.float32)
  hv_ref[...] = hv
  for a in range(4):
    m_ref[a] = jnp.maximum(hv + bond_ref[a][None, :], 0.0)


def _tc2(h, batch3, vn, bond, interpret=False):
  return pl.pallas_call(
      _tc2_body,
      grid=(NBLK,),
      in_specs=[
          pl.BlockSpec((RB, D), lambda i: (i, 0)),
          pl.BlockSpec((1, 1, RB), lambda i: (i, 0, 0)),
          pl.BlockSpec((G, D), lambda i: (0, 0)),
          pl.BlockSpec((8, D), lambda i: (0, 0)),
      ],
      out_specs=[
          pl.BlockSpec((RB, D), lambda i: (i, 0)),
          pl.BlockSpec((4, RB, D), lambda i: (0, i, 0)),
      ],
      out_shape=[
          jax.ShapeDtypeStruct((N_PAD, D), jnp.float32),
          jax.ShapeDtypeStruct((4, N_PAD, D), jnp.float32),
      ],
      interpret=interpret,
  )(h, batch3, vn, bond)


def _tcf_body(hv_ref, p_ref, b_ref, eps_ref, w1_ref, c1_ref, w2_ref, c2_ref,
              pw_ref, pb_ref, out_ref, s_ref, cnt_ref):
  i = pl.program_id(0)

  @pl.when(i == 0)
  def _():
    s_ref[...] = jnp.zeros_like(s_ref)
    cnt_ref[...] = jnp.zeros_like(cnt_ref)

  hv = hv_ref[...]
  z = eps_ref[0, 0] * hv + p_ref[0] + p_ref[1]
  t = jnp.maximum(
      jnp.dot(z, w1_ref[...], preferred_element_type=jnp.float32)
      + c1_ref[...], 0.0)
  zn = jnp.dot(t, w2_ref[...], preferred_element_type=jnp.float32) + c2_ref[...]
  bo = _onehot(b_ref[0, 0, :], G)
  s_ref[...] += lax.dot_general(bo, zn, (((0,), (0,)), ((), ())),
                                preferred_element_type=jnp.float32)
  cnt_ref[...] += jnp.sum(bo, axis=0, keepdims=True)

  @pl.when(i == NBLK - 1)
  def _():
    rc = 1.0 / jnp.maximum(cnt_ref[...], 1.0)
    eye = (lax.broadcasted_iota(jnp.int32, (G, G), 0)
           == lax.broadcasted_iota(jnp.int32, (G, G), 1)).astype(jnp.float32)
    dinv = eye * rc
    hg = jnp.dot(dinv, s_ref[...], preferred_element_type=jnp.float32)
    out_ref[...] = (jnp.dot(hg, pw_ref[...], preferred_element_type=jnp.float32)
                    + pb_ref[...])


def _tcf(hv, partials, batch3, epsb, w1, c1, w2, c2, pw, pb, interpret=False):
  full = lambda shape: pl.BlockSpec(shape, lambda i: tuple(0 for _ in shape))
  return pl.pallas_call(
      _tcf_body,
      grid=(NBLK,),
      in_specs=[
          pl.BlockSpec((RB, D), lambda i: (i, 0)),
          pl.BlockSpec((2, RB, D), lambda i: (0, i, 0)),
          pl.BlockSpec((1, 1, RB), lambda i: (i, 0, 0)),
          full((1, 128)),
          full((D, H)), full((1, H)), full((H, D)), full((1, D)),
          full((D, 128)), full((1, 128)),
      ],
      out_specs=pl.BlockSpec((G, 128), lambda i: (0, 0)),
      out_shape=jax.ShapeDtypeStruct((G, 128), jnp.float32),
      scratch_shapes=[pltpu.VMEM((G, D), jnp.float32),
                      pltpu.VMEM((1, G), jnp.float32)],
      interpret=interpret,
  )(hv, partials, batch3, epsb, w1, c1, w2, c2, pw, pb)


def _sc_body(m_hbm, gidx_hbm, dst_hbm, zero_hbm, out_hbm,
             gidx_v, dst_v, gbuf, acc, sem):
  c = lax.axis_index("c")
  s = lax.axis_index("s")
  w = c * NUM_TILES + s
  pltpu.sync_copy(gidx_hbm.at[w], gidx_v)
  pltpu.sync_copy(dst_hbm.at[w], dst_v)
  pltpu.sync_copy(zero_hbm, acc.at[pl.ds(s * ROWS_PER_TILE, ROWS_PER_TILE)])
  plsc.subcore_barrier()

  def body(j, carry):
    pltpu.async_copy(m_hbm.at[gidx_v.at[j]], gbuf, sem).wait()
    pltpu.sync_copy(gbuf, acc.at[dst_v.at[j]], add=True)
    return carry

  lax.fori_loop(0, E_TILE_CH, body, 0)
  plsc.subcore_barrier()
  pltpu.sync_copy(acc.at[pl.ds(s * ROWS_PER_TILE, ROWS_PER_TILE)],
                  out_hbm.at[c, pl.ds(s * ROWS_PER_TILE, ROWS_PER_TILE)])


@functools.cache
def _make_sc_aggr():
  return pl.kernel(
      _sc_body,
      out_type=jax.ShapeDtypeStruct((NUM_SC, N_PAD, D), jnp.float32),
      mesh=plsc.VectorSubcoreMesh(core_axis_name="c", subcore_axis_name="s",
                                  num_cores=NUM_SC, num_subcores=NUM_TILES),
      scratch_types=[
          pltpu.VMEM((E_TILE_CH, 128), jnp.int32),
          pltpu.VMEM((E_TILE_CH, 128), jnp.int32),
          pltpu.VMEM((128, D), jnp.float32),
          pltpu.VMEM_SHARED((N_PAD, D), jnp.float32),
          pltpu.SemaphoreType.DMA,
      ],
  )


def _sc_aggr(m_flat, gidx_p, dst_p, zero_rows):
  return _make_sc_aggr()(m_flat, gidx_p, dst_p, zero_rows)


def kernel(x, edge_index, edge_attr, batch, node_table, bond_tables, eps,
           mlp_W1, mlp_b1, mlp_bn_g, mlp_bn_b, mlp_bn_m, mlp_bn_v, mlp_W2,
           mlp_b2, bn_g, bn_b, bn_m, bn_v, vn_init,
           vn_W1, vn_b1, vn_bn1_g, vn_bn1_b, vn_bn1_m, vn_bn1_v,
           vn_W2, vn_b2, vn_bn2_g, vn_bn2_b, vn_bn2_m, vn_bn2_v,
           pred_W, pred_b):
  f32 = jnp.float32

  a1 = mlp_bn_g / jnp.sqrt(mlp_bn_v + 1e-5)
  w1f = mlp_W1 * a1[:, None, :]
  c1f = mlp_b1 * a1 + mlp_bn_b - mlp_bn_m * a1
  a2 = bn_g / jnp.sqrt(bn_v + 1e-5)
  w2f = mlp_W2 * a2[:, None, :]
  c2f = mlp_b2 * a2 + bn_b - bn_m * a2
  va1 = vn_bn1_g / jnp.sqrt(vn_bn1_v + 1e-5)
  vw1f = vn_W1 * va1[:, None, :]
  vc1f = vn_b1 * va1 + vn_bn1_b - vn_bn1_m * va1
  va2 = vn_bn2_g / jnp.sqrt(vn_bn2_v + 1e-5)
  vw2f = vn_W2 * va2[:, None, :]
  vc2f = vn_b2 * va2 + vn_bn2_b - vn_bn2_m * va2
  epsb = jnp.broadcast_to((1.0 + eps)[:, None, None], (L, 1, 128))

  tab = jnp.zeros((128, D), f32).at[:node_table.shape[0]].set(node_table)
  bond = jnp.zeros((L, 8, D), f32).at[:, :4].set(bond_tables)
  pw = jnp.zeros((D, 128), f32).at[:, :NC].set(pred_W)
  pb = jnp.zeros((1, 128), f32).at[0, :NC].set(pred_b)
  vn = jnp.broadcast_to(vn_init[None, :], (G, D))

  xi = x.astype(jnp.int32)
  bi = batch.astype(jnp.int32)
  x3 = jnp.full((N_PAD,), 127, jnp.int32).at[:N].set(xi).reshape(NBLK, 1, RB)
  batch3 = jnp.full((N_PAD,), G, jnp.int32).at[:N].set(bi).reshape(NBLK, 1, RB)
  src = edge_index[0].astype(jnp.int32)
  dst = edge_index[1].astype(jnp.int32)
  gidx = edge_attr.astype(jnp.int32) * N_PAD + src
  gidx_p = jnp.zeros((E_PAD,), jnp.int32).at[:E].set(gidx)
  gidx_p = gidx_p.reshape(W_TILES, E_TILE_CH, 128)
  dst_p = jnp.full((E_PAD,), N, jnp.int32).at[:E].set(dst)
  dst_p = dst_p.reshape(W_TILES, E_TILE_CH, 128)
  zero_rows = jnp.zeros((ROWS_PER_TILE, D), f32)

  hv, m = _tc0(x3, batch3, tab, vn, bond[0])
  out = None
  for l in range(L):
    partials = _sc_aggr(m.reshape(4 * N_PAD, D), gidx_p, dst_p, zero_rows)
    if l < L - 1:
      h_new, vn = _tc1(hv, partials, batch3, vn, epsb[l],
                       w1f[l], c1f[l:l + 1], w2f[l], c2f[l:l + 1],
                       vw1f[l], vc1f[l:l + 1], vw2f[l], vc2f[l:l + 1])
      hv, m = _tc2(h_new, batch3, vn, bond[l + 1])
    else:
      out = _tcf(hv, partials, batch3, epsb[l],
                 w1f[l], c1f[l:l + 1], w2f[l], c2f[l:l + 1], pw, pb)
  return out[:, :NC]
